# baseline (speedup 1.0000x reference)
import functools

import jax
import jax.numpy as jnp
from jax import lax
from jax.experimental import pallas as pl
from jax.experimental.pallas import tpu as pltpu
from jax.experimental.pallas import tpu_sc as plsc

EMBED_DIM = 64
NUM_CORES = 2
NUM_SUBCORES = 16
NUM_WORKERS = NUM_CORES * NUM_SUBCORES
NUM_EMB_ROWS = 144

CHUNK = 512
NBUF = 2
LANES = 16


@jax.jit
def _run(idx_flat, table):
    n = idx_flat.shape[0]
    per_w = n // NUM_WORKERS
    steps = per_w // CHUNK
    assert steps % NBUF == 0

    mesh = plsc.VectorSubcoreMesh(
        core_axis_name="c", subcore_axis_name="s",
        num_cores=NUM_CORES, num_subcores=NUM_SUBCORES)

    @functools.partial(
        pl.kernel,
        out_type=jax.ShapeDtypeStruct((n * EMBED_DIM,), jnp.float32),
        mesh=mesh,
        scratch_types=[
            pltpu.VMEM((NBUF, CHUNK), jnp.int32),
            pltpu.VMEM((NBUF, CHUNK * EMBED_DIM), jnp.float32),
            pltpu.VMEM((NUM_EMB_ROWS * EMBED_DIM,), jnp.float32),
            [pltpu.SemaphoreType.DMA] * NBUF,
            [pltpu.SemaphoreType.DMA] * NBUF,
        ],
        compiler_params=pltpu.CompilerParams(
            use_tc_tiling_on_sc=False, needs_layout_passes=False),
    )
    def k(idx_hbm, table_hbm, out_hbm, idx_v, rows_v, table_v,
          sem_idx, sem_out):
        wid = lax.axis_index("s") * NUM_CORES + lax.axis_index("c")
        base = wid * per_w
        pltpu.sync_copy(table_hbm, table_v)

        def idx_copy(step, b):
            return pltpu.make_async_copy(
                idx_hbm.at[pl.ds(base + step * CHUNK, CHUNK)],
                idx_v.at[b], sem_idx[b])

        def out_copy(step, b):
            return pltpu.make_async_copy(
                rows_v.at[b],
                out_hbm.at[pl.ds((base + step * CHUNK) * EMBED_DIM,
                                 CHUNK * EMBED_DIM)],
                sem_out[b])

        for b in range(NBUF):
            idx_copy(b, b).start()

        lane = lax.iota(jnp.int32, LANES)

        def body(g, _):
            for b in range(NBUF):
                step = g * NBUF + b
                idx_copy(step, b).wait()
                @pl.when(g > 0)
                def _drain_prev():
                    out_copy(step - NBUF, b).wait()

                def group(i, _):
                    for r in range(LANES):
                        bidx = plsc.load_gather(
                            idx_v.at[b], [jnp.full((LANES,), 0, jnp.int32)
                                          + (i * LANES + r)])
                        rowbase = bidx * EMBED_DIM + lane
                        dst = (i * LANES + r) * EMBED_DIM
                        for j in range(EMBED_DIM // LANES):
                            val = plsc.load_gather(
                                table_v, [rowbase + j * LANES])
                            rows_v[b, pl.ds(dst + j * LANES, LANES)] = val
                    return _

                lax.fori_loop(0, CHUNK // LANES, group, 0)

                @pl.when(step + NBUF < steps)
                def _prefetch():
                    idx_copy(step + NBUF, b).start()
                out_copy(step, b).start()
            return _

        lax.fori_loop(0, steps // NBUF, body, 0)
        for b in range(NBUF):
            out_copy(steps - NBUF + b, b).wait()

    return k(idx_flat, table.reshape(NUM_EMB_ROWS * EMBED_DIM))


def kernel(channel_indices, table):
    b, f = channel_indices.shape
    idx_flat = channel_indices.reshape(b * f).astype(jnp.int32)
    out = _run(idx_flat, table)
    return out.reshape(b, f, EMBED_DIM)
